# Initial kernel scaffold; baseline (speedup 1.0000x reference)
import jax
import jax.numpy as jnp
from jax.experimental import pallas as pl


def kernel(node_emb, batch):
    raise NotImplementedError("write your pallas kernel here")


# speedup vs baseline: 3.0440x; 3.0440x over previous
import functools

import jax
import jax.numpy as jnp
from jax import lax
from jax.experimental import pallas as pl
from jax.experimental.pallas import tpu as pltpu
from jax.experimental.pallas import tpu_sc as plsc

_NUM_GRAPHS = 64
_N_NODES = 50000
_D = 256
_L = 16
_CH = _D // _L

_B = 80
_NBLOCKS = _N_NODES // _B
_NC = 2
_NS = 16
_NW = _NC * _NS


def _sc_body(emb_hbm, batch_hbm, zsum_hbm, zcnt_hbm,
             sums_hbm, cnts_hbm, stage_v, idx_v, acc_v, cacc_v):
    cid = lax.axis_index("c")
    sid = lax.axis_index("s")
    wid = sid * _NC + cid

    pltpu.sync_copy(zsum_hbm, acc_v)
    pltpu.sync_copy(zcnt_hbm, cacc_v)

    nblk = (_NBLOCKS - 1 - wid) // _NW + 1

    onevec = jnp.ones((_L,), jnp.float32)

    def _node16(t, carry):
        gvec = idx_v[pl.ds(t * _L, _L)]
        for k in range(_L):
            g = gvec[k]
            r = t * _L + k
            for j in range(_CH):
                plsc.addupdate(acc_v.at[g, pl.ds(j * _L, _L)],
                               stage_v[r, pl.ds(j * _L, _L)])
            plsc.addupdate(cacc_v.at[g, :], onevec)
        return carry

    def _block(i, carry):
        b = wid + i * _NW
        base = b * _B
        pltpu.sync_copy(batch_hbm.at[pl.ds(base, _B)], idx_v.at[pl.ds(0, _B)])
        pltpu.sync_copy(emb_hbm.at[pl.ds(base, _B)], stage_v)
        return lax.fori_loop(0, _B // _L, _node16, carry)

    lax.fori_loop(0, nblk, _block, 0)

    pltpu.sync_copy(acc_v, sums_hbm.at[cid, sid])
    pltpu.sync_copy(cacc_v, cnts_hbm.at[cid, sid])


_sc_partial = functools.partial(
    pl.kernel,
    out_type=(
        jax.ShapeDtypeStruct((_NC, _NS, _NUM_GRAPHS, _D), jnp.float32),
        jax.ShapeDtypeStruct((_NC, _NS, _NUM_GRAPHS, _L), jnp.float32),
    ),
    mesh=plsc.VectorSubcoreMesh(core_axis_name="c", subcore_axis_name="s"),
    scratch_types=[
        pltpu.VMEM((_B, _D), jnp.float32),
        pltpu.VMEM((_B + _L,), jnp.int32),
        pltpu.VMEM((_NUM_GRAPHS, _D), jnp.float32),
        pltpu.VMEM((_NUM_GRAPHS, _L), jnp.float32),
    ],
)(_sc_body)


def _combine_body(s_ref, c_ref, o_ref):
    s = jnp.sum(s_ref[...], axis=0)
    c = jnp.sum(c_ref[...], axis=0)
    cnt = jnp.maximum(c, 1.0)[:, 0:1]
    o_ref[...] = s / cnt


def _combine(sums_p, cnts_p):
    return pl.pallas_call(
        _combine_body,
        out_shape=jax.ShapeDtypeStruct((_NUM_GRAPHS, _D), jnp.float32),
    )(sums_p.reshape(_NW, _NUM_GRAPHS, _D), cnts_p.reshape(_NW, _NUM_GRAPHS, _L))


def kernel(node_emb, batch):
    zsum = jnp.zeros((_NUM_GRAPHS, _D), jnp.float32)
    zcnt = jnp.zeros((_NUM_GRAPHS, _L), jnp.float32)
    sums_p, cnts_p = _sc_partial(node_emb, batch, zsum, zcnt)
    return _combine(sums_p, cnts_p)


# speedup vs baseline: 3.8819x; 1.2753x over previous
import functools

import jax
import jax.numpy as jnp
from jax import lax
from jax.experimental import pallas as pl
from jax.experimental.pallas import tpu as pltpu
from jax.experimental.pallas import tpu_sc as plsc

_NUM_GRAPHS = 64
_N_NODES = 50000
_D = 256
_L = 16
_CH = _D // _L

_B = 80
_NBLOCKS = _N_NODES // _B
_NC = 2
_NS = 16
_NW = _NC * _NS


def _sc_body(emb_hbm, batch_hbm, zsum_hbm, zcnt_hbm,
             sums_hbm, cnts_hbm, stage0_v, stage1_v, idx0_v, idx1_v,
             acc_v, cacc_v, sem_i, sem_e):
    cid = lax.axis_index("c")
    sid = lax.axis_index("s")
    wid = sid * _NC + cid

    stages = (stage0_v, stage1_v)
    idxs = (idx0_v, idx1_v)

    pltpu.sync_copy(zsum_hbm, acc_v)
    pltpu.sync_copy(zcnt_hbm, cacc_v)

    nblk = (_NBLOCKS - 1 - wid) // _NW + 1

    onevec = jnp.ones((_L,), jnp.float32)

    def _copies(i, p):
        base = (wid + i * _NW) * _B
        ci = pltpu.make_async_copy(
            batch_hbm.at[pl.ds(base, _B)], idxs[p], sem_i)
        ce = pltpu.make_async_copy(
            emb_hbm.at[pl.ds(base, _B)], stages[p], sem_e)
        return ci, ce

    def _accumulate(p, rows):
        stage_v = stages[p]
        idx_v = idxs[p]

        def _node16(t, carry):
            gvec = idx_v[pl.ds(t * _L, _L)]
            for k in range(_L):
                g = gvec[k]
                r = t * _L + k
                for j in range(_CH):
                    plsc.addupdate(acc_v.at[g, pl.ds(j * _L, _L)],
                                   stage_v[r, pl.ds(j * _L, _L)])
                plsc.addupdate(cacc_v.at[g, :], onevec)
            return carry
        lax.fori_loop(0, rows // _L, _node16, 0)

    ci, ce = _copies(0, 0)
    ci.start()
    ce.start()

    def _block(i, carry):
        for p in (0, 1):
            @pl.when(jnp.bitwise_and(i, 1) == p)
            def _run(p=p):
                ci, ce = _copies(i, p)
                ci.wait()
                ce.wait()

                @pl.when(i + 1 < nblk)
                def _prefetch():
                    ni, ne = _copies(i + 1, 1 - p)
                    ni.start()
                    ne.start()

                _accumulate(p, _B)
        return carry

    lax.fori_loop(0, nblk, _block, 0)

    pltpu.sync_copy(acc_v, sums_hbm.at[cid, sid])
    pltpu.sync_copy(cacc_v, cnts_hbm.at[cid, sid])


_sc_partial = functools.partial(
    pl.kernel,
    out_type=(
        jax.ShapeDtypeStruct((_NC, _NS, _NUM_GRAPHS, _D), jnp.float32),
        jax.ShapeDtypeStruct((_NC, _NS, _NUM_GRAPHS, _L), jnp.float32),
    ),
    mesh=plsc.VectorSubcoreMesh(core_axis_name="c", subcore_axis_name="s"),
    scratch_types=[
        pltpu.VMEM((_B, _D), jnp.float32),
        pltpu.VMEM((_B, _D), jnp.float32),
        pltpu.VMEM((_B,), jnp.int32),
        pltpu.VMEM((_B,), jnp.int32),
        pltpu.VMEM((_NUM_GRAPHS, _D), jnp.float32),
        pltpu.VMEM((_NUM_GRAPHS, _L), jnp.float32),
        pltpu.SemaphoreType.DMA,
        pltpu.SemaphoreType.DMA,
    ],
)(_sc_body)


def _combine_body(s_ref, c_ref, o_ref):
    s = jnp.sum(s_ref[...], axis=0)
    c = jnp.sum(c_ref[...], axis=0)
    cnt = jnp.maximum(c, 1.0)[:, 0:1]
    o_ref[...] = s / cnt


def _combine(sums_p, cnts_p):
    return pl.pallas_call(
        _combine_body,
        out_shape=jax.ShapeDtypeStruct((_NUM_GRAPHS, _D), jnp.float32),
    )(sums_p.reshape(_NW, _NUM_GRAPHS, _D), cnts_p.reshape(_NW, _NUM_GRAPHS, _L))


def kernel(node_emb, batch):
    zsum = jnp.zeros((_NUM_GRAPHS, _D), jnp.float32)
    zcnt = jnp.zeros((_NUM_GRAPHS, _L), jnp.float32)
    sums_p, cnts_p = _sc_partial(node_emb, batch, zsum, zcnt)
    return _combine(sums_p, cnts_p)


# speedup vs baseline: 6.4998x; 1.6744x over previous
import functools

import jax
import jax.numpy as jnp
from jax import lax
from jax.experimental import pallas as pl
from jax.experimental.pallas import tpu as pltpu
from jax.experimental.pallas import tpu_sc as plsc

_NUM_GRAPHS = 64
_N_NODES = 50000
_D = 256
_L = 16
_CH = _D // _L

_B = 80
_NBLOCKS = _N_NODES // _B
_NC = 2
_NS = 16
_NW = _NC * _NS


def _sc_body(emb_hbm, batch_hbm, zsum_hbm, zcnt_hbm,
             sums_hbm, cnts_hbm, stage0_v, stage1_v, idx0_v, idx1_v,
             acc_v, cacc_v, sem_i, sem_e):
    cid = lax.axis_index("c")
    sid = lax.axis_index("s")
    wid = sid * _NC + cid

    stages = (stage0_v, stage1_v)
    idxs = (idx0_v, idx1_v)

    pltpu.sync_copy(zsum_hbm, acc_v)
    pltpu.sync_copy(zcnt_hbm, cacc_v)

    nblk = (_NBLOCKS - 1 - wid) // _NW + 1

    onevec = jnp.ones((_L,), jnp.float32)

    def _copies(i, p):
        base = (wid + i * _NW) * _B
        ci = pltpu.make_async_copy(
            batch_hbm.at[pl.ds(base, _B)], idxs[p], sem_i)
        ce = pltpu.make_async_copy(
            emb_hbm.at[pl.ds(base, _B)], stages[p], sem_e)
        return ci, ce

    zvec = jnp.zeros((_L,), jnp.float32)

    def _accumulate(p, rows):
        stage_v = stages[p]
        idx_v = idxs[p]
        blockvec = jnp.full((_L,), float(rows), jnp.float32)

        g_first = idx_v[pl.ds(0, _L)][0]
        g_last = idx_v[pl.ds(rows - _L, _L)][_L - 1]

        @pl.when(g_first == g_last)
        def _uniform():
            def _row(r, acc):
                return tuple(acc[j] + stage_v[r, pl.ds(j * _L, _L)]
                             for j in range(_CH))
            acc = lax.fori_loop(0, rows, _row, (zvec,) * _CH)
            for j in range(_CH):
                plsc.addupdate(acc_v.at[g_first, pl.ds(j * _L, _L)], acc[j])
            plsc.addupdate(cacc_v.at[g_first, :], blockvec)

        @pl.when(g_first != g_last)
        def _mixed():
            def _node16(t, carry):
                gvec = idx_v[pl.ds(t * _L, _L)]
                for k in range(_L):
                    g = gvec[k]
                    r = t * _L + k
                    for j in range(_CH):
                        plsc.addupdate(acc_v.at[g, pl.ds(j * _L, _L)],
                                       stage_v[r, pl.ds(j * _L, _L)])
                    plsc.addupdate(cacc_v.at[g, :], onevec)
                return carry
            lax.fori_loop(0, rows // _L, _node16, 0)

    ci, ce = _copies(0, 0)
    ci.start()
    ce.start()

    def _block(i, carry):
        for p in (0, 1):
            @pl.when(jnp.bitwise_and(i, 1) == p)
            def _run(p=p):
                ci, ce = _copies(i, p)
                ci.wait()
                ce.wait()

                @pl.when(i + 1 < nblk)
                def _prefetch():
                    ni, ne = _copies(i + 1, 1 - p)
                    ni.start()
                    ne.start()

                _accumulate(p, _B)
        return carry

    lax.fori_loop(0, nblk, _block, 0)

    pltpu.sync_copy(acc_v, sums_hbm.at[cid, sid])
    pltpu.sync_copy(cacc_v, cnts_hbm.at[cid, sid])


_sc_partial = functools.partial(
    pl.kernel,
    out_type=(
        jax.ShapeDtypeStruct((_NC, _NS, _NUM_GRAPHS, _D), jnp.float32),
        jax.ShapeDtypeStruct((_NC, _NS, _NUM_GRAPHS, _L), jnp.float32),
    ),
    mesh=plsc.VectorSubcoreMesh(core_axis_name="c", subcore_axis_name="s"),
    scratch_types=[
        pltpu.VMEM((_B, _D), jnp.float32),
        pltpu.VMEM((_B, _D), jnp.float32),
        pltpu.VMEM((_B,), jnp.int32),
        pltpu.VMEM((_B,), jnp.int32),
        pltpu.VMEM((_NUM_GRAPHS, _D), jnp.float32),
        pltpu.VMEM((_NUM_GRAPHS, _L), jnp.float32),
        pltpu.SemaphoreType.DMA,
        pltpu.SemaphoreType.DMA,
    ],
)(_sc_body)


def _combine_body(s_ref, c_ref, o_ref):
    s = jnp.sum(s_ref[...], axis=0)
    c = jnp.sum(c_ref[...], axis=0)
    cnt = jnp.maximum(c, 1.0)[:, 0:1]
    o_ref[...] = s / cnt


def _combine(sums_p, cnts_p):
    return pl.pallas_call(
        _combine_body,
        out_shape=jax.ShapeDtypeStruct((_NUM_GRAPHS, _D), jnp.float32),
    )(sums_p.reshape(_NW, _NUM_GRAPHS, _D), cnts_p.reshape(_NW, _NUM_GRAPHS, _L))


def kernel(node_emb, batch):
    zsum = jnp.zeros((_NUM_GRAPHS, _D), jnp.float32)
    zcnt = jnp.zeros((_NUM_GRAPHS, _L), jnp.float32)
    sums_p, cnts_p = _sc_partial(node_emb, batch, zsum, zcnt)
    return _combine(sums_p, cnts_p)


# speedup vs baseline: 8.2058x; 1.2625x over previous
import functools

import jax
import jax.numpy as jnp
from jax import lax
from jax.experimental import pallas as pl
from jax.experimental.pallas import tpu as pltpu
from jax.experimental.pallas import tpu_sc as plsc

_NUM_GRAPHS = 64
_N_NODES = 50000
_D = 256
_L = 16
_CH = _D // _L

_B = 80
_NBLOCKS = _N_NODES // _B
_NC = 2
_NS = 16
_NW = _NC * _NS


def _sc_body(emb_hbm, batch_hbm,
             sums_hbm, cnts_hbm, stage0_v, stage1_v, idx0_v, idx1_v,
             acc_v, cacc_v, sem_i, sem_e):
    cid = lax.axis_index("c")
    sid = lax.axis_index("s")
    wid = sid * _NC + cid

    stages = (stage0_v, stage1_v)
    idxs = (idx0_v, idx1_v)

    nblk = (_NBLOCKS - 1 - wid) // _NW + 1

    onevec = jnp.ones((_L,), jnp.float32)

    def _copies(i, p):
        base = (wid + i * _NW) * _B
        ci = pltpu.make_async_copy(
            batch_hbm.at[pl.ds(base, _B)], idxs[p], sem_i)
        ce = pltpu.make_async_copy(
            emb_hbm.at[pl.ds(base, _B)], stages[p], sem_e)
        return ci, ce

    zvec = jnp.zeros((_L,), jnp.float32)

    def _accumulate(p, rows):
        stage_v = stages[p]
        idx_v = idxs[p]
        blockvec = jnp.full((_L,), float(rows), jnp.float32)
        chunkvec = jnp.full((_L,), float(_L), jnp.float32)

        g_first = idx_v[pl.ds(0, _L)][0]
        g_last = idx_v[pl.ds(rows - _L, _L)][_L - 1]

        @pl.when(g_first == g_last)
        def _uniform():
            def _row(r, acc):
                return tuple(acc[j] + stage_v[r, pl.ds(j * _L, _L)]
                             for j in range(_CH))
            acc = lax.fori_loop(0, rows, _row, (zvec,) * _CH)
            for j in range(_CH):
                plsc.addupdate(acc_v.at[g_first, pl.ds(j * _L, _L)], acc[j])
            plsc.addupdate(cacc_v.at[g_first, :], blockvec)

        @pl.when(g_first != g_last)
        def _mixed():
            def _chunk16(t, carry):
                gvec = idx_v[pl.ds(t * _L, _L)]
                c_first = gvec[0]
                c_last = gvec[_L - 1]

                @pl.when(c_first == c_last)
                def _chunk_uniform():
                    def _row(r, acc):
                        return tuple(acc[j] + stage_v[r, pl.ds(j * _L, _L)]
                                     for j in range(_CH))
                    acc = lax.fori_loop(t * _L, (t + 1) * _L, _row,
                                        (zvec,) * _CH)
                    for j in range(_CH):
                        plsc.addupdate(acc_v.at[c_first, pl.ds(j * _L, _L)],
                                       acc[j])
                    plsc.addupdate(cacc_v.at[c_first, :], chunkvec)

                @pl.when(c_first != c_last)
                def _chunk_mixed():
                    for k in range(_L):
                        g = gvec[k]
                        r = t * _L + k
                        for j in range(_CH):
                            plsc.addupdate(acc_v.at[g, pl.ds(j * _L, _L)],
                                           stage_v[r, pl.ds(j * _L, _L)])
                        plsc.addupdate(cacc_v.at[g, :], onevec)
                return carry
            lax.fori_loop(0, rows // _L, _chunk16, 0)

    ci, ce = _copies(0, 0)
    ci.start()
    ce.start()

    def _zero_row(i, carry):
        for j in range(_CH):
            acc_v[i, pl.ds(j * _L, _L)] = zvec
        cacc_v[i, :] = zvec
        return carry
    lax.fori_loop(0, _NUM_GRAPHS, _zero_row, 0)

    def _block(i, carry):
        for p in (0, 1):
            @pl.when(jnp.bitwise_and(i, 1) == p)
            def _run(p=p):
                ci, ce = _copies(i, p)
                ci.wait()
                ce.wait()

                @pl.when(i + 1 < nblk)
                def _prefetch():
                    ni, ne = _copies(i + 1, 1 - p)
                    ni.start()
                    ne.start()

                _accumulate(p, _B)
        return carry

    lax.fori_loop(0, nblk, _block, 0)

    pltpu.sync_copy(acc_v, sums_hbm.at[cid, sid])
    pltpu.sync_copy(cacc_v, cnts_hbm.at[cid, sid])


_sc_partial = functools.partial(
    pl.kernel,
    out_type=(
        jax.ShapeDtypeStruct((_NC, _NS, _NUM_GRAPHS, _D), jnp.float32),
        jax.ShapeDtypeStruct((_NC, _NS, _NUM_GRAPHS, _L), jnp.float32),
    ),
    mesh=plsc.VectorSubcoreMesh(core_axis_name="c", subcore_axis_name="s"),
    scratch_types=[
        pltpu.VMEM((_B, _D), jnp.float32),
        pltpu.VMEM((_B, _D), jnp.float32),
        pltpu.VMEM((_B,), jnp.int32),
        pltpu.VMEM((_B,), jnp.int32),
        pltpu.VMEM((_NUM_GRAPHS, _D), jnp.float32),
        pltpu.VMEM((_NUM_GRAPHS, _L), jnp.float32),
        pltpu.SemaphoreType.DMA,
        pltpu.SemaphoreType.DMA,
    ],
)(_sc_body)


def _combine_body(s_ref, c_ref, o_ref):
    s = jnp.sum(s_ref[...], axis=0)
    c = jnp.sum(c_ref[...], axis=0)
    cnt = jnp.maximum(c, 1.0)[:, 0:1]
    o_ref[...] = s / cnt


def _combine(sums_p, cnts_p):
    return pl.pallas_call(
        _combine_body,
        out_shape=jax.ShapeDtypeStruct((_NUM_GRAPHS, _D), jnp.float32),
    )(sums_p.reshape(_NW, _NUM_GRAPHS, _D), cnts_p.reshape(_NW, _NUM_GRAPHS, _L))


def kernel(node_emb, batch):
    sums_p, cnts_p = _sc_partial(node_emb, batch)
    return _combine(sums_p, cnts_p)


# speedup vs baseline: 8.6109x; 1.0494x over previous
import functools

import jax
import jax.numpy as jnp
from jax import lax
from jax.experimental import pallas as pl
from jax.experimental.pallas import tpu as pltpu
from jax.experimental.pallas import tpu_sc as plsc

_NUM_GRAPHS = 64
_N_NODES = 50000
_D = 256
_L = 16
_CH = _D // _L

_B = 80
_NBLOCKS = _N_NODES // _B
_NC = 2
_NS = 16
_NW = _NC * _NS


_NBUF = 4


def _sc_body(emb_hbm, batch_hbm,
             sums_hbm, cnts_hbm, stage0_v, stage1_v, stage2_v, stage3_v,
             idx0_v, idx1_v, idx2_v, idx3_v,
             acc_v, cacc_v, sem0, sem1, sem2, sem3):
    cid = lax.axis_index("c")
    sid = lax.axis_index("s")
    wid = sid * _NC + cid

    stages = (stage0_v, stage1_v, stage2_v, stage3_v)
    idxs = (idx0_v, idx1_v, idx2_v, idx3_v)
    sems = (sem0, sem1, sem2, sem3)

    nblk = (_NBLOCKS - 1 - wid) // _NW + 1

    onevec = jnp.ones((_L,), jnp.float32)

    def _copies(i, p):
        base = (wid + i * _NW) * _B
        ci = pltpu.make_async_copy(
            batch_hbm.at[pl.ds(base, _B)], idxs[p], sems[p])
        ce = pltpu.make_async_copy(
            emb_hbm.at[pl.ds(base, _B)], stages[p], sems[p])
        return ci, ce

    zvec = jnp.zeros((_L,), jnp.float32)

    def _accumulate(p, rows):
        stage_v = stages[p]
        idx_v = idxs[p]
        blockvec = jnp.full((_L,), float(rows), jnp.float32)
        chunkvec = jnp.full((_L,), float(_L), jnp.float32)

        g_first = idx_v[pl.ds(0, _L)][0]
        g_last = idx_v[pl.ds(rows - _L, _L)][_L - 1]

        @pl.when(g_first == g_last)
        def _uniform():
            def _row(r, acc):
                return tuple(acc[j] + stage_v[r, pl.ds(j * _L, _L)]
                             for j in range(_CH))
            acc = lax.fori_loop(0, rows, _row, (zvec,) * _CH)
            for j in range(_CH):
                plsc.addupdate(acc_v.at[g_first, pl.ds(j * _L, _L)], acc[j])
            plsc.addupdate(cacc_v.at[g_first, :], blockvec)

        @pl.when(g_first != g_last)
        def _mixed():
            def _chunk16(t, carry):
                gvec = idx_v[pl.ds(t * _L, _L)]
                c_first = gvec[0]
                c_last = gvec[_L - 1]

                @pl.when(c_first == c_last)
                def _chunk_uniform():
                    def _row(r, acc):
                        return tuple(acc[j] + stage_v[r, pl.ds(j * _L, _L)]
                                     for j in range(_CH))
                    acc = lax.fori_loop(t * _L, (t + 1) * _L, _row,
                                        (zvec,) * _CH)
                    for j in range(_CH):
                        plsc.addupdate(acc_v.at[c_first, pl.ds(j * _L, _L)],
                                       acc[j])
                    plsc.addupdate(cacc_v.at[c_first, :], chunkvec)

                @pl.when(c_first != c_last)
                def _chunk_mixed():
                    for k in range(_L):
                        g = gvec[k]
                        r = t * _L + k
                        for j in range(_CH):
                            plsc.addupdate(acc_v.at[g, pl.ds(j * _L, _L)],
                                           stage_v[r, pl.ds(j * _L, _L)])
                        plsc.addupdate(cacc_v.at[g, :], onevec)
                return carry
            lax.fori_loop(0, rows // _L, _chunk16, 0)

    for q in range(_NBUF - 1):
        @pl.when(q < nblk)
        def _prime(q=q):
            ci, ce = _copies(q, q)
            ci.start()
            ce.start()

    def _zero_row(i, carry):
        for j in range(_CH):
            acc_v[i, pl.ds(j * _L, _L)] = zvec
        cacc_v[i, :] = zvec
        return carry
    lax.fori_loop(0, _NUM_GRAPHS, _zero_row, 0)

    def _block(i, carry):
        for p in range(_NBUF):
            @pl.when(jnp.bitwise_and(i, _NBUF - 1) == p)
            def _run(p=p):
                ci, ce = _copies(i, p)
                ci.wait()
                ce.wait()

                @pl.when(i + _NBUF - 1 < nblk)
                def _prefetch():
                    ni, ne = _copies(i + _NBUF - 1,
                                     (p + _NBUF - 1) % _NBUF)
                    ni.start()
                    ne.start()

                _accumulate(p, _B)
        return carry

    lax.fori_loop(0, nblk, _block, 0)

    pltpu.sync_copy(acc_v, sums_hbm.at[cid, sid])
    pltpu.sync_copy(cacc_v, cnts_hbm.at[cid, sid])


_sc_partial = functools.partial(
    pl.kernel,
    out_type=(
        jax.ShapeDtypeStruct((_NC, _NS, _NUM_GRAPHS, _D), jnp.float32),
        jax.ShapeDtypeStruct((_NC, _NS, _NUM_GRAPHS, _L), jnp.float32),
    ),
    mesh=plsc.VectorSubcoreMesh(core_axis_name="c", subcore_axis_name="s"),
    scratch_types=(
        [pltpu.VMEM((_B, _D), jnp.float32)] * 4
        + [pltpu.VMEM((_B,), jnp.int32)] * 4
        + [
            pltpu.VMEM((_NUM_GRAPHS, _D), jnp.float32),
            pltpu.VMEM((_NUM_GRAPHS, _L), jnp.float32),
        ]
        + [pltpu.SemaphoreType.DMA] * 4
    ),
)(_sc_body)


def _combine_body(s_ref, c_ref, o_ref):
    s = jnp.sum(s_ref[...], axis=0)
    c = jnp.sum(c_ref[...], axis=0)
    cnt = jnp.maximum(c, 1.0)[:, 0:1]
    o_ref[...] = s / cnt


def _combine(sums_p, cnts_p):
    return pl.pallas_call(
        _combine_body,
        out_shape=jax.ShapeDtypeStruct((_NUM_GRAPHS, _D), jnp.float32),
    )(sums_p.reshape(_NW, _NUM_GRAPHS, _D), cnts_p.reshape(_NW, _NUM_GRAPHS, _L))


def kernel(node_emb, batch):
    sums_p, cnts_p = _sc_partial(node_emb, batch)
    return _combine(sums_p, cnts_p)
